# baseline (speedup 1.0000x reference)
import functools

import jax
import jax.numpy as jnp
import numpy as np
from jax import lax
from jax.experimental import pallas as pl
from jax.experimental.pallas import tpu as pltpu
from jax.experimental.pallas import tpu_sc as plsc

_B = 16
_Q = 300
_H = 8
_D = 32
_PTS = 12
_N = _B * _Q
_SEQ = 8400
_HID = 256
_T = 4 * _H * _PTS
_ROWS_BLK = 600
_GRID = _N // _ROWS_BLK

_SPATIAL = ((80, 80), (40, 40), (20, 20))
_NUM_PTS = (4, 4, 4)

_lvl_of_p = np.repeat(np.arange(3), _NUM_PTS)
_W12 = np.array([_SPATIAL[l][1] for l in _lvl_of_p], np.int32)
_H12 = np.array([_SPATIAL[l][0] for l in _lvl_of_p], np.int32)
_BASE12 = np.array(
    [int(np.cumsum([0] + [h * w for h, w in _SPATIAL])[l]) for l in _lvl_of_p],
    np.int32)
_WLI = np.tile(_W12, _H)[None]
_WLF = _WLI.astype(np.float32)
_HLI = np.tile(_H12, _H)[None]
_HLF = _HLI.astype(np.float32)
_BASEI = np.tile(_BASE12, _H)[None]
_HLANE = (np.arange(_H * _PTS)[None] // _PTS).astype(np.int32)


def _prep_body(x_ref, rp_ref, wx_ref, wy_ref, wa_ref, bx_ref, by_ref, ba_ref,
               nps_ref, wlf_ref, hlf_ref, wli_ref, base_ref, hlane_ref,
               aw_ref, w_ref, idx_ref):
    x = x_ref[...]
    sx = jnp.dot(x, wx_ref[...], preferred_element_type=jnp.float32) + bx_ref[...]
    sy = jnp.dot(x, wy_ref[...], preferred_element_type=jnp.float32) + by_ref[...]
    sa = jnp.dot(x, wa_ref[...], preferred_element_type=jnp.float32) + ba_ref[...]

    parts = []
    for h in range(_H):
        sl = sa[:, h * _PTS:(h + 1) * _PTS]
        m = jnp.max(sl, axis=1, keepdims=True)
        e = jnp.exp(sl - m)
        parts.append(e / jnp.sum(e, axis=1, keepdims=True))
    aw = jnp.concatenate(parts, axis=1)
    aw_ref[...] = aw

    rp = rp_ref[...]
    cx = rp[:, 0:1]
    cy = rp[:, 1:2]
    rw = rp[:, 2:3]
    rh = rp[:, 3:4]
    scale = nps_ref[...] * 0.5
    wlf = wlf_ref[...]
    hlf = hlf_ref[...]
    locx = cx + sx * (rw * scale)
    locy = cy + sy * (rh * scale)
    X = locx * wlf - 0.5
    Y = locy * hlf - 0.5
    x0 = jnp.floor(X)
    fx = X - x0
    y0 = jnp.floor(Y)
    fy = Y - y0

    pid = pl.program_id(0)
    n = pid * _ROWS_BLK + lax.broadcasted_iota(jnp.int32, (_ROWS_BLK, 1), 0)
    boff = (n // _Q) * _SEQ

    wli = wli_ref[...]
    basei = base_ref[...]
    hlane = hlane_ref[...]
    for c, (dx, dy) in enumerate(((0, 0), (1, 0), (0, 1), (1, 1))):
        xf = x0 + dx
        yf = y0 + dy
        valid = ((xf >= 0.0) & (xf <= wlf - 1.0) &
                 (yf >= 0.0) & (yf <= hlf - 1.0))
        ii = jnp.clip(xf, 0.0, wlf - 1.0).astype(jnp.int32)
        jj = jnp.clip(yf, 0.0, hlf - 1.0).astype(jnp.int32)
        g = (boff + jj * wli + ii + basei) * _H + hlane
        wxc = fx if dx else 1.0 - fx
        wyc = fy if dy else 1.0 - fy
        wgt = aw * wxc * wyc * jnp.where(valid, 1.0, 0.0)
        npts = _H * _PTS
        w_ref[:, c * npts:(c + 1) * npts] = wgt
        idx_ref[:, c * npts:(c + 1) * npts] = g


def _prep_call(x, rp, wx, wy, wa, bx, by, ba, nps96):
    npts = _H * _PTS
    row_spec = lambda shape: pl.BlockSpec(shape, lambda i: (i, 0))
    full_spec = lambda shape: pl.BlockSpec(shape, lambda i: (0, 0))
    return pl.pallas_call(
        _prep_body,
        grid=(_GRID,),
        in_specs=[
            row_spec((_ROWS_BLK, _HID)),
            row_spec((_ROWS_BLK, 4)),
            full_spec((_HID, npts)),
            full_spec((_HID, npts)),
            full_spec((_HID, npts)),
            full_spec((1, npts)),
            full_spec((1, npts)),
            full_spec((1, npts)),
            full_spec((1, npts)),
            full_spec((1, npts)),
            full_spec((1, npts)),
            full_spec((1, npts)),
            full_spec((1, npts)),
            full_spec((1, npts)),
        ],
        out_specs=[
            row_spec((_ROWS_BLK, npts)),
            row_spec((_ROWS_BLK, _T)),
            row_spec((_ROWS_BLK, _T)),
        ],
        out_shape=[
            jax.ShapeDtypeStruct((_N, npts), jnp.float32),
            jax.ShapeDtypeStruct((_N, _T), jnp.float32),
            jax.ShapeDtypeStruct((_N, _T), jnp.int32),
        ],
    )(x, rp, wx, wy, wa, bx, by, ba, nps96,
      jnp.asarray(_WLF), jnp.asarray(_HLF), jnp.asarray(_WLI),
      jnp.asarray(_BASEI), jnp.asarray(_HLANE))


_NW = 32
_NPW = _N // _NW
_GRP = 3
_NGRP = _NPW // _GRP
_GT = _GRP * _T
_CHK = _GT // 128


def _sc_body(tbl_hbm, idx_hbm, w_hbm, out_hbm,
             idx_a, idx_b, w_a, w_b, rows_a, rows_b, out_a, out_b,
             isem_a, isem_b, gsem_a, gsem_b, osem_a, osem_b):
    wid = lax.axis_index("s") * 2 + lax.axis_index("c")
    row0 = wid * _NPW

    bufs = ((idx_a, w_a, rows_a, out_a, isem_a, gsem_a, osem_a),
            (idx_b, w_b, rows_b, out_b, isem_b, gsem_b, osem_b))

    def fire_idxw(g, p):
        idx_v, w_v, _, _, isem, _, _ = bufs[p]
        off = (row0 + g * _GRP) * _T
        pltpu.async_copy(idx_hbm.at[pl.ds(off, _GT)], idx_v, isem)
        pltpu.async_copy(w_hbm.at[pl.ds(off, _GT)],
                         w_v.at[pl.ds(0, _GT)], isem)

    def drain_idxw(p):
        idx_v, w_v, _, _, isem, _, _ = bufs[p]
        pltpu.make_async_copy(idx_hbm.at[pl.ds(0, _GT)], idx_v, isem).wait()
        pltpu.make_async_copy(w_hbm.at[pl.ds(0, _GT)],
                              w_v.at[pl.ds(0, _GT)], isem).wait()

    def fire_gathers(p):
        idx_v, _, rows_v, _, _, gsem, _ = bufs[p]
        for j in range(_CHK):
            pltpu.async_copy(tbl_hbm.at[idx_v.at[pl.ds(j * 128, 128)]],
                             rows_v.at[pl.ds(j * 128, 128)], gsem)

    def drain_gathers(p):
        idx_v, _, rows_v, _, _, gsem, _ = bufs[p]
        for j in range(_CHK):
            pltpu.make_async_copy(
                tbl_hbm.at[idx_v.at[pl.ds(j * 128, 128)]],
                rows_v.at[pl.ds(j * 128, 128)], gsem).wait()

    def compute_and_store(g, p):
        _, w_v, rows_v, out_v, _, _, osem = bufs[p]

        def row(r, carry):
            gi = r // _H
            h = r - gi * _H
            acc0 = jnp.zeros((16,), jnp.float32)
            acc1 = jnp.zeros((16,), jnp.float32)
            for c in range(4):
                jb = gi * _T + c * (_H * _PTS) + h * _PTS
                wv = w_v[pl.ds(jb, 16)]
                for pp in range(_PTS):
                    w = wv[pp]
                    ve, vo = plsc.unpack(rows_v[jb + pp, :],
                                         format=plsc.PackFormat.INTERLEAVED)
                    acc0 = acc0 + w * ve
                    acc1 = acc1 + w * vo
            out_v[r, 0:16] = acc0
            out_v[r, 16:32] = acc1
            return carry

        lax.fori_loop(0, _GRP * _H, row, 0)
        pltpu.async_copy(
            out_v, out_hbm.at[pl.ds((row0 + g * _GRP) * _H, _GRP * _H)], osem)

    def drain_out(p):
        _, _, _, out_v, _, _, osem = bufs[p]
        pltpu.make_async_copy(out_v, out_hbm.at[pl.ds(0, _GRP * _H)],
                              osem).wait()

    def group_step(g, p):
        @pl.when(g + 1 < _NGRP)
        def _():
            drain_idxw(1 - p)
            fire_gathers(1 - p)
        drain_gathers(p)

        @pl.when(g >= 2)
        def _():
            drain_out(p)
        compute_and_store(g, p)

        @pl.when(g + 2 < _NGRP)
        def _():
            fire_idxw(g + 2, p)

    fire_idxw(0, 0)
    drain_idxw(0)
    fire_gathers(0)
    fire_idxw(1, 1)

    def pair(t, carry):
        group_step(2 * t, 0)
        group_step(2 * t + 1, 1)
        return carry

    lax.fori_loop(0, _NGRP // 2, pair, 0)
    drain_out(0)
    drain_out(1)


def _sc_call(tbl, idx1, wflat):
    mesh = plsc.VectorSubcoreMesh(core_axis_name="c", subcore_axis_name="s")
    return pl.kernel(
        _sc_body,
        mesh=mesh,
        compiler_params=pltpu.CompilerParams(use_tc_tiling_on_sc=False,
                                             needs_layout_passes=False),
        out_type=jax.ShapeDtypeStruct((_N * _H, _D), jnp.float32),
        scratch_types=[
            pltpu.VMEM((_GT,), jnp.int32),
            pltpu.VMEM((_GT,), jnp.int32),
            pltpu.VMEM((_GT + 16,), jnp.float32),
            pltpu.VMEM((_GT + 16,), jnp.float32),
            pltpu.VMEM((_GT, _D), jnp.bfloat16),
            pltpu.VMEM((_GT, _D), jnp.bfloat16),
            pltpu.VMEM((_GRP * _H, _D), jnp.float32),
            pltpu.VMEM((_GRP * _H, _D), jnp.float32),
            pltpu.SemaphoreType.DMA,
            pltpu.SemaphoreType.DMA,
            pltpu.SemaphoreType.DMA,
            pltpu.SemaphoreType.DMA,
            pltpu.SemaphoreType.DMA,
            pltpu.SemaphoreType.DMA,
        ],
    )(tbl, idx1, wflat)


def kernel(hidden_states, encoder_hidden_states, reference_points,
           spatial_shapes, W_off, b_off, W_attn, b_attn, num_points_scale):
    del spatial_shapes
    npts = _H * _PTS
    x = hidden_states.reshape(_N, _HID)
    rp = reference_points.reshape(_N, 4)
    wo = W_off.reshape(_HID, npts, 2)
    wx = wo[:, :, 0]
    wy = wo[:, :, 1]
    bo = b_off.reshape(npts, 2)
    bx = bo[:, 0][None]
    by = bo[:, 1][None]
    ba = b_attn[None]
    nps96 = jnp.tile(num_points_scale, _H)[None]

    aw, wts, idx = _prep_call(x, rp, wx, wy, W_attn, bx, by, ba, nps96)

    tbl = encoder_hidden_states.astype(jnp.bfloat16).reshape(_B * _SEQ * _H, _D)
    idx1 = idx.reshape(_N * _T)
    wflat = wts.reshape(_N * _T)
    outv = _sc_call(tbl, idx1, wflat)

    outv = jnp.transpose(outv.reshape(_N * _H, 2, 16), (0, 2, 1))
    out = outv.reshape(_B, _Q, _H * _D)
    aw_out = aw.reshape(_B, _Q, _H, _PTS)
    return out, aw_out


# speedup vs baseline: 3.4274x; 3.4274x over previous
import functools

import jax
import jax.numpy as jnp
import numpy as np
from jax import lax
from jax.experimental import pallas as pl
from jax.experimental.pallas import tpu as pltpu
from jax.experimental.pallas import tpu_sc as plsc

_B = 16
_Q = 300
_H = 8
_D = 32
_PTS = 12
_N = _B * _Q
_SEQ = 8400
_HID = 256
_T = 4 * _H * _PTS
_ROWS_BLK = 600
_GRID = _N // _ROWS_BLK

_SPATIAL = ((80, 80), (40, 40), (20, 20))
_NUM_PTS = (4, 4, 4)

_lvl_of_p = np.repeat(np.arange(3), _NUM_PTS)
_W12 = np.array([_SPATIAL[l][1] for l in _lvl_of_p], np.int32)
_H12 = np.array([_SPATIAL[l][0] for l in _lvl_of_p], np.int32)
_BASE12 = np.array(
    [int(np.cumsum([0] + [h * w for h, w in _SPATIAL])[l]) for l in _lvl_of_p],
    np.int32)
_WLI = np.tile(_W12, _H)[None]
_WLF = _WLI.astype(np.float32)
_HLI = np.tile(_H12, _H)[None]
_HLF = _HLI.astype(np.float32)
_BASEI = np.tile(_BASE12, _H)[None]
_HLANE = (np.arange(_H * _PTS)[None] // _PTS).astype(np.int32)


def _prep_body(x_ref, rp_ref, wx_ref, wy_ref, wa_ref, bx_ref, by_ref, ba_ref,
               nps_ref, wlf_ref, hlf_ref, wli_ref, base_ref, hlane_ref,
               aw_ref, w_ref, idx_ref):
    x = x_ref[...]
    sx = jnp.dot(x, wx_ref[...], preferred_element_type=jnp.float32) + bx_ref[...]
    sy = jnp.dot(x, wy_ref[...], preferred_element_type=jnp.float32) + by_ref[...]
    sa = jnp.dot(x, wa_ref[...], preferred_element_type=jnp.float32) + ba_ref[...]

    parts = []
    for h in range(_H):
        sl = sa[:, h * _PTS:(h + 1) * _PTS]
        m = jnp.max(sl, axis=1, keepdims=True)
        e = jnp.exp(sl - m)
        parts.append(e / jnp.sum(e, axis=1, keepdims=True))
    aw = jnp.concatenate(parts, axis=1)
    aw_ref[...] = aw

    rp = rp_ref[...]
    cx = rp[:, 0:1]
    cy = rp[:, 1:2]
    rw = rp[:, 2:3]
    rh = rp[:, 3:4]
    scale = nps_ref[...] * 0.5
    wlf = wlf_ref[...]
    hlf = hlf_ref[...]
    locx = cx + sx * (rw * scale)
    locy = cy + sy * (rh * scale)
    X = locx * wlf - 0.5
    Y = locy * hlf - 0.5
    x0 = jnp.floor(X)
    fx = X - x0
    y0 = jnp.floor(Y)
    fy = Y - y0

    pid = pl.program_id(0)
    n = pid * _ROWS_BLK + lax.broadcasted_iota(jnp.int32, (_ROWS_BLK, 1), 0)
    boff = (n // _Q) * _SEQ

    wli = wli_ref[...]
    basei = base_ref[...]
    hlane = hlane_ref[...]
    for c, (dx, dy) in enumerate(((0, 0), (1, 0), (0, 1), (1, 1))):
        xf = x0 + dx
        yf = y0 + dy
        valid = ((xf >= 0.0) & (xf <= wlf - 1.0) &
                 (yf >= 0.0) & (yf <= hlf - 1.0))
        ii = jnp.clip(xf, 0.0, wlf - 1.0).astype(jnp.int32)
        jj = jnp.clip(yf, 0.0, hlf - 1.0).astype(jnp.int32)
        g = (boff + jj * wli + ii + basei) * _H + hlane
        wxc = fx if dx else 1.0 - fx
        wyc = fy if dy else 1.0 - fy
        wgt = aw * wxc * wyc * jnp.where(valid, 1.0, 0.0)
        npts = _H * _PTS
        w_ref[:, c * npts:(c + 1) * npts] = wgt
        idx_ref[:, c * npts:(c + 1) * npts] = g


def _prep_call(x, rp, wx, wy, wa, bx, by, ba, nps96):
    npts = _H * _PTS
    row_spec = lambda shape: pl.BlockSpec(shape, lambda i: (i, 0))
    full_spec = lambda shape: pl.BlockSpec(shape, lambda i: (0, 0))
    return pl.pallas_call(
        _prep_body,
        grid=(_GRID,),
        in_specs=[
            row_spec((_ROWS_BLK, _HID)),
            row_spec((_ROWS_BLK, 4)),
            full_spec((_HID, npts)),
            full_spec((_HID, npts)),
            full_spec((_HID, npts)),
            full_spec((1, npts)),
            full_spec((1, npts)),
            full_spec((1, npts)),
            full_spec((1, npts)),
            full_spec((1, npts)),
            full_spec((1, npts)),
            full_spec((1, npts)),
            full_spec((1, npts)),
            full_spec((1, npts)),
        ],
        out_specs=[
            row_spec((_ROWS_BLK, npts)),
            row_spec((_ROWS_BLK, _T)),
            row_spec((_ROWS_BLK, _T)),
        ],
        out_shape=[
            jax.ShapeDtypeStruct((_N, npts), jnp.float32),
            jax.ShapeDtypeStruct((_N, _T), jnp.float32),
            jax.ShapeDtypeStruct((_N, _T), jnp.int32),
        ],
    )(x, rp, wx, wy, wa, bx, by, ba, nps96,
      jnp.asarray(_WLF), jnp.asarray(_HLF), jnp.asarray(_WLI),
      jnp.asarray(_BASEI), jnp.asarray(_HLANE))


_NW = 32
_NPW = _N // _NW
_GRP = 3
_NGRP = _NPW // _GRP
_GT = _GRP * _T
_CHK = _GT // 128


def _sc_body(tbl_hbm, idx_hbm, w_hbm, out_hbm,
             idx_a, idx_b, w_a, w_b, rows_a, rows_b, out_a, out_b,
             isem_a, isem_b, gsem_a, gsem_b, osem_a, osem_b):
    wid = lax.axis_index("s") * 2 + lax.axis_index("c")
    row0 = wid * _NPW

    bufs = ((idx_a, w_a, rows_a, out_a, isem_a, gsem_a, osem_a),
            (idx_b, w_b, rows_b, out_b, isem_b, gsem_b, osem_b))

    def fire_idxw(g, p):
        idx_v, w_v, _, _, isem, _, _ = bufs[p]
        off = (row0 + g * _GRP) * _T
        pltpu.async_copy(idx_hbm.at[pl.ds(off, _GT)], idx_v, isem)
        pltpu.async_copy(w_hbm.at[pl.ds(off, _GT)],
                         w_v.at[pl.ds(0, _GT)], isem)

    def drain_idxw(p):
        idx_v, w_v, _, _, isem, _, _ = bufs[p]
        pltpu.make_async_copy(idx_hbm.at[pl.ds(0, _GT)], idx_v, isem).wait()
        pltpu.make_async_copy(w_hbm.at[pl.ds(0, _GT)],
                              w_v.at[pl.ds(0, _GT)], isem).wait()

    def fire_gathers(p):
        idx_v, _, rows_v, _, _, gsem, _ = bufs[p]
        for j in range(_CHK):
            pltpu.async_copy(tbl_hbm.at[idx_v.at[pl.ds(j * 128, 128)]],
                             rows_v.at[pl.ds(j * 128, 128)], gsem)

    def drain_gathers(p):
        idx_v, _, rows_v, _, _, gsem, _ = bufs[p]
        for j in range(_CHK):
            pltpu.make_async_copy(
                tbl_hbm.at[idx_v.at[pl.ds(j * 128, 128)]],
                rows_v.at[pl.ds(j * 128, 128)], gsem).wait()

    def compute_and_store(g, p):
        _, w_v, rows_v, out_v, _, _, osem = bufs[p]

        def row(r, carry):
            gi = r // _H
            h = r - gi * _H
            acc0 = jnp.zeros((16,), jnp.float32)
            acc1 = jnp.zeros((16,), jnp.float32)
            for c in range(4):
                jb = gi * _T + c * (_H * _PTS) + h * _PTS
                wv = w_v[pl.ds(jb, 16)]
                for pp in range(_PTS):
                    w = wv[pp]
                    acc0 = acc0 + w * rows_v[jb + pp, 0:16]
                    acc1 = acc1 + w * rows_v[jb + pp, 16:32]
            out_v[r, 0:16] = acc0
            out_v[r, 16:32] = acc1
            return carry

        lax.fori_loop(0, _GRP * _H, row, 0)
        pltpu.async_copy(
            out_v, out_hbm.at[pl.ds((row0 + g * _GRP) * _H, _GRP * _H)], osem)

    def drain_out(p):
        _, _, _, out_v, _, _, osem = bufs[p]
        pltpu.make_async_copy(out_v, out_hbm.at[pl.ds(0, _GRP * _H)],
                              osem).wait()

    def group_step(g, p):
        @pl.when(g + 1 < _NGRP)
        def _():
            drain_idxw(1 - p)
            fire_gathers(1 - p)
        drain_gathers(p)

        @pl.when(g >= 2)
        def _():
            drain_out(p)
        compute_and_store(g, p)

        @pl.when(g + 2 < _NGRP)
        def _():
            fire_idxw(g + 2, p)

    fire_idxw(0, 0)
    drain_idxw(0)
    fire_gathers(0)
    fire_idxw(1, 1)

    def pair(t, carry):
        group_step(2 * t, 0)
        group_step(2 * t + 1, 1)
        return carry

    lax.fori_loop(0, _NGRP // 2, pair, 0)
    drain_out(0)
    drain_out(1)


def _sc_call(tbl, idx1, wflat):
    mesh = plsc.VectorSubcoreMesh(core_axis_name="c", subcore_axis_name="s")
    return pl.kernel(
        _sc_body,
        mesh=mesh,
        compiler_params=pltpu.CompilerParams(use_tc_tiling_on_sc=False),
        out_type=jax.ShapeDtypeStruct((_N * _H, _D), jnp.float32),
        scratch_types=[
            pltpu.VMEM((_GT,), jnp.int32),
            pltpu.VMEM((_GT,), jnp.int32),
            pltpu.VMEM((_GT + 16,), jnp.float32),
            pltpu.VMEM((_GT + 16,), jnp.float32),
            pltpu.VMEM((_GT, _D), jnp.float32),
            pltpu.VMEM((_GT, _D), jnp.float32),
            pltpu.VMEM((_GRP * _H, _D), jnp.float32),
            pltpu.VMEM((_GRP * _H, _D), jnp.float32),
            pltpu.SemaphoreType.DMA,
            pltpu.SemaphoreType.DMA,
            pltpu.SemaphoreType.DMA,
            pltpu.SemaphoreType.DMA,
            pltpu.SemaphoreType.DMA,
            pltpu.SemaphoreType.DMA,
        ],
    )(tbl, idx1, wflat)


def kernel(hidden_states, encoder_hidden_states, reference_points,
           spatial_shapes, W_off, b_off, W_attn, b_attn, num_points_scale):
    del spatial_shapes
    npts = _H * _PTS
    x = hidden_states.reshape(_N, _HID)
    rp = reference_points.reshape(_N, 4)
    wo = W_off.reshape(_HID, npts, 2)
    wx = wo[:, :, 0]
    wy = wo[:, :, 1]
    bo = b_off.reshape(npts, 2)
    bx = bo[:, 0][None]
    by = bo[:, 1][None]
    ba = b_attn[None]
    nps96 = jnp.tile(num_points_scale, _H)[None]

    aw, wts, idx = _prep_call(x, rp, wx, wy, W_attn, bx, by, ba, nps96)

    tbl = encoder_hidden_states.reshape(_B * _SEQ * _H, _D)
    idx1 = idx.reshape(_N * _T)
    wflat = wts.reshape(_N * _T)
    outv = _sc_call(tbl, idx1, wflat)

    out = outv.reshape(_B, _Q, _H * _D)
    aw_out = aw.reshape(_B, _Q, _H, _PTS)
    return out, aw_out


# speedup vs baseline: 3.4339x; 1.0019x over previous
import jax
import jax.numpy as jnp
import numpy as np
from jax import lax
from jax.experimental import pallas as pl
from jax.experimental.pallas import tpu as pltpu
from jax.experimental.pallas import tpu_sc as plsc

_B = 16
_Q = 300
_H = 8
_D = 32
_PTS = 12
_N = _B * _Q
_SEQ = 8400
_HID = 256
_T = 4 * _H * _PTS
_ROWS_BLK = 600
_GRID = _N // _ROWS_BLK

_SPATIAL = ((80, 80), (40, 40), (20, 20))
_NUM_PTS = (4, 4, 4)

_lvl_of_p = np.repeat(np.arange(3), _NUM_PTS)
_W12 = np.array([_SPATIAL[l][1] for l in _lvl_of_p], np.int32)
_H12 = np.array([_SPATIAL[l][0] for l in _lvl_of_p], np.int32)
_BASE12 = np.array(
    [int(np.cumsum([0] + [h * w for h, w in _SPATIAL])[l]) for l in _lvl_of_p],
    np.int32)
_WLI = np.tile(_W12, _H)[None]
_WLF = _WLI.astype(np.float32)
_HLI = np.tile(_H12, _H)[None]
_HLF = _HLI.astype(np.float32)
_BASEI = np.tile(_BASE12, _H)[None]
_HLANE = (np.arange(_H * _PTS)[None] // _PTS).astype(np.int32)


def _prep_body(x_ref, rp_ref, wx_ref, wy_ref, wa_ref, bx_ref, by_ref, ba_ref,
               nps_ref, wlf_ref, hlf_ref, wli_ref, base_ref, hlane_ref,
               aw_ref, w_ref, idx_ref):
    x = x_ref[...]
    sx = jnp.dot(x, wx_ref[...], preferred_element_type=jnp.float32) + bx_ref[...]
    sy = jnp.dot(x, wy_ref[...], preferred_element_type=jnp.float32) + by_ref[...]
    sa = jnp.dot(x, wa_ref[...], preferred_element_type=jnp.float32) + ba_ref[...]

    parts = []
    for h in range(_H):
        sl = sa[:, h * _PTS:(h + 1) * _PTS]
        m = jnp.max(sl, axis=1, keepdims=True)
        e = jnp.exp(sl - m)
        parts.append(e / jnp.sum(e, axis=1, keepdims=True))
    aw = jnp.concatenate(parts, axis=1)
    aw_ref[...] = aw

    rp = rp_ref[...]
    cx = rp[:, 0:1]
    cy = rp[:, 1:2]
    rw = rp[:, 2:3]
    rh = rp[:, 3:4]
    scale = nps_ref[...] * 0.5
    wlf = wlf_ref[...]
    hlf = hlf_ref[...]
    locx = cx + sx * (rw * scale)
    locy = cy + sy * (rh * scale)
    X = locx * wlf - 0.5
    Y = locy * hlf - 0.5
    x0 = jnp.floor(X)
    fx = X - x0
    y0 = jnp.floor(Y)
    fy = Y - y0

    pid = pl.program_id(0)
    n = pid * _ROWS_BLK + lax.broadcasted_iota(jnp.int32, (_ROWS_BLK, 1), 0)
    boff = (n // _Q) * _SEQ

    wli = wli_ref[...]
    basei = base_ref[...]
    hlane = hlane_ref[...]
    for c, (dx, dy) in enumerate(((0, 0), (1, 0), (0, 1), (1, 1))):
        xf = x0 + dx
        yf = y0 + dy
        valid = ((xf >= 0.0) & (xf <= wlf - 1.0) &
                 (yf >= 0.0) & (yf <= hlf - 1.0))
        ii = jnp.clip(xf, 0.0, wlf - 1.0).astype(jnp.int32)
        jj = jnp.clip(yf, 0.0, hlf - 1.0).astype(jnp.int32)
        g = (boff + jj * wli + ii + basei) * _H + hlane
        wxc = fx if dx else 1.0 - fx
        wyc = fy if dy else 1.0 - fy
        wgt = aw * wxc * wyc * jnp.where(valid, 1.0, 0.0)
        npts = _H * _PTS
        w_ref[:, c * npts:(c + 1) * npts] = wgt
        idx_ref[:, c * npts:(c + 1) * npts] = g


def _prep_call(x, rp, wx, wy, wa, bx, by, ba, nps96):
    npts = _H * _PTS
    row_spec = lambda shape: pl.BlockSpec(shape, lambda i: (i, 0))
    full_spec = lambda shape: pl.BlockSpec(shape, lambda i: (0, 0))
    return pl.pallas_call(
        _prep_body,
        grid=(_GRID,),
        in_specs=[
            row_spec((_ROWS_BLK, _HID)),
            row_spec((_ROWS_BLK, 4)),
            full_spec((_HID, npts)),
            full_spec((_HID, npts)),
            full_spec((_HID, npts)),
            full_spec((1, npts)),
            full_spec((1, npts)),
            full_spec((1, npts)),
            full_spec((1, npts)),
            full_spec((1, npts)),
            full_spec((1, npts)),
            full_spec((1, npts)),
            full_spec((1, npts)),
            full_spec((1, npts)),
        ],
        out_specs=[
            row_spec((_ROWS_BLK, npts)),
            row_spec((_ROWS_BLK, _T)),
            row_spec((_ROWS_BLK, _T)),
        ],
        out_shape=[
            jax.ShapeDtypeStruct((_N, npts), jnp.float32),
            jax.ShapeDtypeStruct((_N, _T), jnp.float32),
            jax.ShapeDtypeStruct((_N, _T), jnp.int32),
        ],
    )(x, rp, wx, wy, wa, bx, by, ba, nps96,
      jnp.asarray(_WLF), jnp.asarray(_HLF), jnp.asarray(_WLI),
      jnp.asarray(_BASEI), jnp.asarray(_HLANE))


_NW = 32
_NPW = _N // _NW
_GRP = 3
_NGRP = _NPW // _GRP
_GT = _GRP * _T
_CHK = _GT // 128


def _sc_body(tbl_hbm, idx_hbm, w_hbm, out_hbm,
             idx_a, idx_b, w_a, w_b, rows_a, rows_b, out_a, out_b,
             isem_a, isem_b, gsem_a, gsem_b, osem_a, osem_b):
    wid = lax.axis_index("s") * 2 + lax.axis_index("c")
    row0 = wid * _NPW

    bufs = ((idx_a, w_a, rows_a, out_a, isem_a, gsem_a, osem_a),
            (idx_b, w_b, rows_b, out_b, isem_b, gsem_b, osem_b))

    def fire_idxw(g, p):
        idx_v, w_v, _, _, isem, _, _ = bufs[p]
        off = (row0 + g * _GRP) * _T
        pltpu.async_copy(idx_hbm.at[pl.ds(off, _GT)], idx_v, isem)
        pltpu.async_copy(w_hbm.at[pl.ds(off, _GT)],
                         w_v.at[pl.ds(0, _GT)], isem)

    def drain_idxw(p):
        idx_v, w_v, _, _, isem, _, _ = bufs[p]
        pltpu.make_async_copy(idx_hbm.at[pl.ds(0, _GT)], idx_v, isem).wait()
        pltpu.make_async_copy(w_hbm.at[pl.ds(0, _GT)],
                              w_v.at[pl.ds(0, _GT)], isem).wait()

    def fire_gathers(p):
        idx_v, _, rows_v, _, _, gsem, _ = bufs[p]
        pltpu.async_copy(tbl_hbm.at[idx_v], rows_v, gsem)

    def drain_gathers(p):
        idx_v, _, rows_v, _, _, gsem, _ = bufs[p]
        pltpu.make_async_copy(tbl_hbm.at[idx_v], rows_v, gsem).wait()

    def compute_and_store(g, p):
        _, w_v, rows_v, out_v, _, _, osem = bufs[p]

        def row(r, carry):
            gi = r // _H
            h = r - gi * _H
            acc0 = jnp.zeros((16,), jnp.float32)
            acc1 = jnp.zeros((16,), jnp.float32)
            for c in range(4):
                jb = gi * _T + c * (_H * _PTS) + h * _PTS
                wv = w_v[pl.ds(jb, 16)]
                for pp in range(_PTS):
                    w = wv[pp]
                    acc0 = acc0 + w * rows_v[jb + pp, 0:16]
                    acc1 = acc1 + w * rows_v[jb + pp, 16:32]
            out_v[r, 0:16] = acc0
            out_v[r, 16:32] = acc1
            return carry

        lax.fori_loop(0, _GRP * _H, row, 0)
        pltpu.async_copy(
            out_v, out_hbm.at[pl.ds((row0 + g * _GRP) * _H, _GRP * _H)], osem)

    def drain_out(p):
        _, _, _, out_v, _, _, osem = bufs[p]
        pltpu.make_async_copy(out_v, out_hbm.at[pl.ds(0, _GRP * _H)],
                              osem).wait()

    def group_step(g, p):
        @pl.when(g + 1 < _NGRP)
        def _():
            drain_idxw(1 - p)
            fire_gathers(1 - p)
        drain_gathers(p)

        @pl.when(g >= 2)
        def _():
            drain_out(p)
        compute_and_store(g, p)

        @pl.when(g + 2 < _NGRP)
        def _():
            fire_idxw(g + 2, p)

    fire_idxw(0, 0)
    drain_idxw(0)
    fire_gathers(0)
    fire_idxw(1, 1)

    def pair(t, carry):
        group_step(2 * t, 0)
        group_step(2 * t + 1, 1)
        return carry

    lax.fori_loop(0, _NGRP // 2, pair, 0)
    drain_out(0)
    drain_out(1)


def _sc_call(tbl, idx1, wflat):
    mesh = plsc.VectorSubcoreMesh(core_axis_name="c", subcore_axis_name="s")
    return pl.kernel(
        _sc_body,
        mesh=mesh,
        compiler_params=pltpu.CompilerParams(use_tc_tiling_on_sc=False),
        out_type=jax.ShapeDtypeStruct((_N * _H, _D), jnp.float32),
        scratch_types=[
            pltpu.VMEM((_GT,), jnp.int32),
            pltpu.VMEM((_GT,), jnp.int32),
            pltpu.VMEM((_GT + 16,), jnp.float32),
            pltpu.VMEM((_GT + 16,), jnp.float32),
            pltpu.VMEM((_GT, _D), jnp.float32),
            pltpu.VMEM((_GT, _D), jnp.float32),
            pltpu.VMEM((_GRP * _H, _D), jnp.float32),
            pltpu.VMEM((_GRP * _H, _D), jnp.float32),
            pltpu.SemaphoreType.DMA,
            pltpu.SemaphoreType.DMA,
            pltpu.SemaphoreType.DMA,
            pltpu.SemaphoreType.DMA,
            pltpu.SemaphoreType.DMA,
            pltpu.SemaphoreType.DMA,
        ],
    )(tbl, idx1, wflat)


def kernel(hidden_states, encoder_hidden_states, reference_points,
           spatial_shapes, W_off, b_off, W_attn, b_attn, num_points_scale):
    del spatial_shapes
    npts = _H * _PTS
    x = hidden_states.reshape(_N, _HID)
    rp = reference_points.reshape(_N, 4)
    wo = W_off.reshape(_HID, npts, 2)
    wx = wo[:, :, 0]
    wy = wo[:, :, 1]
    bo = b_off.reshape(npts, 2)
    bx = bo[:, 0][None]
    by = bo[:, 1][None]
    ba = b_attn[None]
    nps96 = jnp.tile(num_points_scale, _H)[None]

    aw, wts, idx = _prep_call(x, rp, wx, wy, W_attn, bx, by, ba, nps96)

    tbl = encoder_hidden_states.reshape(_B * _SEQ * _H, _D)
    idx1 = idx.reshape(_N * _T)
    wflat = wts.reshape(_N * _T)
    outv = _sc_call(tbl, idx1, wflat)

    out = outv.reshape(_B, _Q, _H * _D)
    aw_out = aw.reshape(_B, _Q, _H, _PTS)
    return out, aw_out
